# baseline (speedup 1.0000x reference)
import jax
import jax.numpy as jnp
from jax.experimental import pallas as pl
from jax.experimental.pallas import tpu as pltpu

_OUT_DIM = 4
_PACK = 8


def _mlp_kernel(x_ref, w1_ref, b1_ref, w2_ref, b2_ref, o_ref):
    h = jnp.dot(x_ref[...], w1_ref[...], preferred_element_type=jnp.float32)
    h = jnp.maximum(h + b1_ref[...], 0.0)
    logits = jnp.dot(h, w2_ref[...], preferred_element_type=jnp.float32)
    o_ref[...] = (logits + b2_ref[...]).astype(o_ref.dtype)


def kernel(x, w1p, b1p, w2p, b2p):
    B, in_dim = x.shape

    xr = jnp.reshape(x, (B // _PACK, _PACK * in_dim))
    N = xr.shape[0]

    w2s = w2p[:, :_OUT_DIM]
    eye = jnp.eye(_PACK, dtype=jnp.float32)
    w1e = jnp.kron(eye, w1p)
    b1e = jnp.tile(b1p, (1, _PACK))
    w2e = jnp.kron(eye, w2s)
    b2e = jnp.tile(b2p[:, :_OUT_DIM], (1, _PACK))

    tbr = 4096
    n_tiles = N // tbr

    out = pl.pallas_call(
        _mlp_kernel,
        out_shape=jax.ShapeDtypeStruct((N, _PACK * _OUT_DIM), jnp.float32),
        grid=(n_tiles,),
        in_specs=[
            pl.BlockSpec((tbr, _PACK * in_dim), lambda i: (i, 0)),
            pl.BlockSpec(w1e.shape, lambda i: (0, 0)),
            pl.BlockSpec(b1e.shape, lambda i: (0, 0)),
            pl.BlockSpec(w2e.shape, lambda i: (0, 0)),
            pl.BlockSpec(b2e.shape, lambda i: (0, 0)),
        ],
        out_specs=pl.BlockSpec((tbr, _PACK * _OUT_DIM), lambda i: (i, 0)),
        compiler_params=pltpu.CompilerParams(
            dimension_semantics=("parallel",)
        ),
    )(xr, w1e, b1e, w2e, b2e)

    return jnp.reshape(out, (B, _OUT_DIM))


# speedup vs baseline: 1.7223x; 1.7223x over previous
import jax
import jax.numpy as jnp
from jax.experimental import pallas as pl
from jax.experimental.pallas import tpu as pltpu

_OUT_DIM = 4
_SLAB = 128


def _mlp_kernel(x_ref, w1_ref, b1_ref, w2_ref, b2_ref, o_ref):
    tbs = x_ref.shape[0]
    x = x_ref[...].reshape(tbs * _SLAB, x_ref.shape[2])
    h = jnp.dot(x, w1_ref[...], preferred_element_type=jnp.float32)
    h = jnp.maximum(h + b1_ref[...], 0.0)
    logits = jnp.dot(h, w2_ref[...], preferred_element_type=jnp.float32)
    logits = logits + b2_ref[...]
    o_ref[...] = logits.reshape(tbs, _SLAB, _OUT_DIM).astype(o_ref.dtype)


def kernel(x, w1p, b1p, w2p, b2p):
    B, in_dim = x.shape
    w2s = w2p[:, :_OUT_DIM]
    b2s = b2p[:, :_OUT_DIM]

    xv = jnp.reshape(x, (B // _SLAB, _SLAB, in_dim))
    n_slabs = xv.shape[0]
    tbs = 64
    n_tiles = n_slabs // tbs

    out = pl.pallas_call(
        _mlp_kernel,
        out_shape=jax.ShapeDtypeStruct((n_slabs, _SLAB, _OUT_DIM), jnp.float32),
        grid=(n_tiles,),
        in_specs=[
            pl.BlockSpec((tbs, _SLAB, in_dim), lambda i: (i, 0, 0)),
            pl.BlockSpec(w1p.shape, lambda i: (0, 0)),
            pl.BlockSpec(b1p.shape, lambda i: (0, 0)),
            pl.BlockSpec(w2s.shape, lambda i: (0, 0)),
            pl.BlockSpec(b2s.shape, lambda i: (0, 0)),
        ],
        out_specs=pl.BlockSpec((tbs, _SLAB, _OUT_DIM), lambda i: (i, 0, 0)),
        compiler_params=pltpu.CompilerParams(
            dimension_semantics=("parallel",)
        ),
    )(xv, w1p, b1p, w2s, b2s)

    return jnp.reshape(out, (B, _OUT_DIM))


# speedup vs baseline: 1.7418x; 1.0113x over previous
import jax
import jax.numpy as jnp
from jax.experimental import pallas as pl
from jax.experimental.pallas import tpu as pltpu

_OUT_DIM = 4
_SLAB = 128


def _mlp_kernel(x_ref, w1_ref, b1_ref, w2_ref, b2_ref, o_ref):
    tbs = x_ref.shape[0]
    x = x_ref[...].reshape(tbs * _SLAB, x_ref.shape[2])
    h = jnp.dot(x, w1_ref[...], preferred_element_type=jnp.float32)
    h = jnp.maximum(h + b1_ref[...], 0.0)
    logits = jnp.dot(h, w2_ref[...], preferred_element_type=jnp.float32)
    logits = logits + b2_ref[...]
    o_ref[...] = logits.reshape(tbs, _SLAB, _OUT_DIM).astype(o_ref.dtype)


def kernel(x, w1p, b1p, w2p, b2p):
    B, in_dim = x.shape
    w2s = w2p[:, :_OUT_DIM]
    b2s = b2p[:, :_OUT_DIM]

    xv = jnp.reshape(x, (B // _SLAB, _SLAB, in_dim))
    n_slabs = xv.shape[0]
    tbs = 128
    n_tiles = n_slabs // tbs

    out = pl.pallas_call(
        _mlp_kernel,
        out_shape=jax.ShapeDtypeStruct((n_slabs, _SLAB, _OUT_DIM), jnp.float32),
        grid=(n_tiles,),
        in_specs=[
            pl.BlockSpec((tbs, _SLAB, in_dim), lambda i: (i, 0, 0)),
            pl.BlockSpec(w1p.shape, lambda i: (0, 0)),
            pl.BlockSpec(b1p.shape, lambda i: (0, 0)),
            pl.BlockSpec(w2s.shape, lambda i: (0, 0)),
            pl.BlockSpec(b2s.shape, lambda i: (0, 0)),
        ],
        out_specs=pl.BlockSpec((tbs, _SLAB, _OUT_DIM), lambda i: (i, 0, 0)),
        compiler_params=pltpu.CompilerParams(
            dimension_semantics=("parallel",)
        ),
    )(xv, w1p, b1p, w2s, b2s)

    return jnp.reshape(out, (B, _OUT_DIM))
